# baseline (speedup 1.0000x reference)
import functools

import jax
import jax.numpy as jnp
from jax import lax
from jax.experimental import pallas as pl
from jax.experimental.pallas import tpu as pltpu
from jax.experimental.pallas import tpu_sc as plsc

N = 100000
E = 1600000
G = 1000
NUM_TOKENS = 512
BASE = 48
EDGE = 4
HID = 64
INV_SQRT_NN = 0.25

NC = 2
NS = 16
LANES = 16

CHUNK = 128
BLK = 8
EB = CHUNK * BLK
E_PAD = 1605632
E_ROWS = E_PAD // CHUNK
N_PAD = 102400
N_ACC = 100352
G_ACC = 1024
PANEL = 16

_mesh = plsc.VectorSubcoreMesh(core_axis_name="c", subcore_axis_name="s")


def _silu(x):
  return x / (1.0 + jnp.exp(-x))


MBLK = 2
MCHUNK = 256
MEB = MBLK * MCHUNK
M_ROWS = E_PAD // MCHUNK


def _msg_body(src_hbm, dst_hbm, zeros_hbm, xp0, xp1, xp2,
              out_p0, out_p1, out_p2a, out_p2b,
              acc, ts_src, ts_dst, ts_rows, sem, ssem, isem):
  c = lax.axis_index("c")
  s = lax.axis_index("s")
  rows_per_tile = N_ACC // NS

  def zero_acc():
    pltpu.sync_copy(zeros_hbm.at[pl.ds(0, rows_per_tile), :],
                    acc.at[pl.ds(s * rows_per_tile, rows_per_tile), :])

  def run_pass(xp, row_lo, nblocks):
    pltpu.sync_copy(src_hbm.at[pl.ds(row_lo, MBLK), :], ts_src.at[0])
    pltpu.sync_copy(dst_hbm.at[pl.ds(row_lo, MBLK), :], ts_dst.at[0])

    def block(b, carry):
      buf = lax.rem(b, 2)
      obuf = 1 - buf
      r0n = row_lo + (b + 1) * MBLK
      for j in range(MBLK):
        pltpu.async_copy(xp.at[ts_src.at[buf, j]],
                         ts_rows.at[buf, pl.ds(j * MCHUNK, MCHUNK), :], sem)

      @pl.when(b >= 1)
      def _():
        for j in range(MBLK):
          pltpu.make_async_copy(
              ts_rows.at[obuf, pl.ds(j * MCHUNK, MCHUNK), :],
              acc.at[ts_dst.at[obuf, j]], ssem).wait()

      @pl.when(b + 1 < nblocks)
      def _():
        pltpu.async_copy(src_hbm.at[pl.ds(r0n, MBLK), :], ts_src.at[obuf],
                         isem)
        pltpu.async_copy(dst_hbm.at[pl.ds(r0n, MBLK), :], ts_dst.at[obuf],
                         isem)

      for j in range(MBLK):
        pltpu.make_async_copy(xp.at[ts_src.at[buf, j]],
                              ts_rows.at[buf, pl.ds(j * MCHUNK, MCHUNK), :],
                              sem).wait()
      for j in range(MBLK):
        pltpu.async_copy(ts_rows.at[buf, pl.ds(j * MCHUNK, MCHUNK), :],
                         acc.at[ts_dst.at[buf, j]], ssem, add=True)

      @pl.when(b + 1 < nblocks)
      def _():
        pltpu.make_async_copy(src_hbm.at[pl.ds(r0n, MBLK), :],
                              ts_src.at[obuf], isem).wait()
        pltpu.make_async_copy(dst_hbm.at[pl.ds(r0n, MBLK), :],
                              ts_dst.at[obuf], isem).wait()
      return carry
    lax.fori_loop(0, nblocks, block, 0)
    lbuf = (nblocks - 1) % 2
    for j in range(MBLK):
      pltpu.make_async_copy(ts_rows.at[lbuf, pl.ds(j * MCHUNK, MCHUNK), :],
                            acc.at[ts_dst.at[lbuf, j]], ssem).wait()

  def drain(out_ref):
    dr = N_ACC // NS
    pltpu.sync_copy(acc.at[pl.ds(s * dr, dr), :],
                    out_ref.at[pl.ds(s * dr, dr), :])

  full_rows = M_ROWS // NS
  half_rows = M_ROWS // (2 * NS)

  zero_acc()
  plsc.subcore_barrier()

  @pl.when(c == 0)
  def _():
    run_pass(xp0, s * full_rows, full_rows // MBLK)

  @pl.when(c == 1)
  def _():
    run_pass(xp1, s * full_rows, full_rows // MBLK)

  plsc.subcore_barrier()

  @pl.when(c == 0)
  def _():
    drain(out_p0)

  @pl.when(c == 1)
  def _():
    drain(out_p1)

  plsc.subcore_barrier()

  zero_acc()
  plsc.subcore_barrier()
  run_pass(xp2, c * (M_ROWS // 2) + s * half_rows, half_rows // MBLK)
  plsc.subcore_barrier()

  @pl.when(c == 0)
  def _():
    drain(out_p2a)

  @pl.when(c == 1)
  def _():
    drain(out_p2b)


_msg_kernel = functools.partial(
    pl.kernel,
    _msg_body,
    out_type=[jax.ShapeDtypeStruct((N_ACC, PANEL), jnp.float32)] * 4,
    mesh=_mesh,
    compiler_params=pltpu.CompilerParams(use_tc_tiling_on_sc=False),
    scratch_types=[
        pltpu.VMEM_SHARED((N_ACC, PANEL), jnp.float32),
        pltpu.VMEM((2, MBLK, MCHUNK), jnp.int32),
        pltpu.VMEM((2, MBLK, MCHUNK), jnp.int32),
        pltpu.VMEM((2, MEB, PANEL), jnp.float32),
        pltpu.SemaphoreType.DMA,
        pltpu.SemaphoreType.DMA,
        pltpu.SemaphoreType.DMA,
    ],
)()


def _embed_body(t0_hbm, t1_hbm, t2_hbm, node_hbm, ea_hbm, dst_hbm, zeros_hbm,
                x0p0, x0p1, x0p2,
                ea_a0, ea_a1, ea_a2, ea_a3, ea_b0, ea_b1, ea_b2, ea_b3,
                acc0, acc1, acc2, acc3, ts_tok, ts_out0, ts_out1, ts_out2,
                ts_eat, ts_dst, sem, wsem, ssem):
  c = lax.axis_index("c")
  s = lax.axis_index("s")
  w = c * NS + s
  rows_per_tile = N_ACC // NS
  accs = (acc0, acc1, acc2, acc3)

  for a in accs:
    pltpu.sync_copy(zeros_hbm, a.at[pl.ds(s * rows_per_tile, rows_per_tile)])
  pltpu.sync_copy(node_hbm.at[w], ts_tok)
  plsc.subcore_barrier()

  tabs = (t0_hbm, t1_hbm, t2_hbm)
  xps = (x0p0, x0p1, x0p2)
  outs = (ts_out0, ts_out1, ts_out2)

  def chunk_body(cb, carry):
    buf = lax.rem(cb, 2)
    obuf = 1 - buf
    base = w * 3200 + cb * CHUNK
    for p in range(3):
      pltpu.async_copy(tabs[p].at[ts_tok.at[cb]], outs[p].at[buf], sem)

    @pl.when(cb >= 1)
    def _():
      for p in range(3):
        pltpu.make_async_copy(
            outs[p].at[obuf],
            xps[p].at[pl.ds(base - CHUNK, CHUNK), :], wsem).wait()
    for p in range(3):
      pltpu.make_async_copy(tabs[p].at[ts_tok.at[cb]], outs[p].at[buf],
                            sem).wait()
    for p in range(3):
      pltpu.async_copy(outs[p].at[buf], xps[p].at[pl.ds(base, CHUNK), :],
                       wsem)
    return carry
  lax.fori_loop(0, 25, chunk_body, 0)
  for p in range(3):
    pltpu.make_async_copy(outs[p].at[0],
                          xps[p].at[pl.ds(w * 3200 + 24 * CHUNK, CHUNK), :],
                          wsem).wait()

  half_rows = E_ROWS // (2 * NS)
  row_lo = c * (E_ROWS // 2) + s * half_rows

  pltpu.sync_copy(dst_hbm.at[pl.ds(row_lo, BLK), :], ts_dst.at[0])
  for k in range(EDGE):
    pltpu.sync_copy(ea_hbm.at[pl.ds(k * E_ROWS + row_lo, BLK), :],
                    ts_eat.at[0, pl.ds(k * BLK, BLK), :])

  def block(b, carry):
    buf = lax.rem(b, 2)
    obuf = 1 - buf
    r0n = row_lo + (b + 1) * BLK
    nblocks = half_rows // BLK
    for k in range(EDGE):
      for j in range(BLK):
        pltpu.async_copy(ts_eat.at[buf, k * BLK + j],
                         accs[k].at[ts_dst.at[buf, j]], ssem, add=True)

    @pl.when(b + 1 < nblocks)
    def _():
      pltpu.async_copy(dst_hbm.at[pl.ds(r0n, BLK), :], ts_dst.at[obuf],
                       sem)
      for k in range(EDGE):
        pltpu.async_copy(ea_hbm.at[pl.ds(k * E_ROWS + r0n, BLK), :],
                         ts_eat.at[obuf, pl.ds(k * BLK, BLK), :], sem)
      pltpu.make_async_copy(dst_hbm.at[pl.ds(r0n, BLK), :],
                            ts_dst.at[obuf], sem).wait()
      for k in range(EDGE):
        pltpu.make_async_copy(ea_hbm.at[pl.ds(k * E_ROWS + r0n, BLK), :],
                              ts_eat.at[obuf, pl.ds(k * BLK, BLK), :],
                              sem).wait()
    for k in range(EDGE):
      for j in range(BLK):
        pltpu.make_async_copy(ts_eat.at[buf, k * BLK + j],
                              accs[k].at[ts_dst.at[buf, j]], ssem).wait()
    return carry
  lax.fori_loop(0, half_rows // BLK, block, 0)

  plsc.subcore_barrier()
  dr = N_ACC // NS
  outs_a = (ea_a0, ea_a1, ea_a2, ea_a3)
  outs_b = (ea_b0, ea_b1, ea_b2, ea_b3)

  @pl.when(c == 0)
  def _():
    for k in range(EDGE):
      pltpu.sync_copy(accs[k].at[pl.ds(s * dr, dr)],
                      outs_a[k].at[pl.ds(s * dr, dr)])

  @pl.when(c == 1)
  def _():
    for k in range(EDGE):
      pltpu.sync_copy(accs[k].at[pl.ds(s * dr, dr)],
                      outs_b[k].at[pl.ds(s * dr, dr)])


_embed_kernel = functools.partial(
    pl.kernel,
    _embed_body,
    out_type=[jax.ShapeDtypeStruct((N_PAD, PANEL), jnp.float32)] * 3
    + [jax.ShapeDtypeStruct((N_ACC,), jnp.float32)] * 8,
    mesh=_mesh,
    compiler_params=pltpu.CompilerParams(use_tc_tiling_on_sc=False),
    scratch_types=[
        pltpu.VMEM_SHARED((N_ACC,), jnp.float32),
        pltpu.VMEM_SHARED((N_ACC,), jnp.float32),
        pltpu.VMEM_SHARED((N_ACC,), jnp.float32),
        pltpu.VMEM_SHARED((N_ACC,), jnp.float32),
        pltpu.VMEM((32, CHUNK), jnp.int32),
        pltpu.VMEM((2, CHUNK, PANEL), jnp.float32),
        pltpu.VMEM((2, CHUNK, PANEL), jnp.float32),
        pltpu.VMEM((2, CHUNK, PANEL), jnp.float32),
        pltpu.VMEM((2, BLK * EDGE, CHUNK), jnp.float32),
        pltpu.VMEM((2, BLK, CHUNK), jnp.int32),
        pltpu.SemaphoreType.DMA,
        pltpu.SemaphoreType.DMA,
        pltpu.SemaphoreType.DMA,
    ],
)()


def _pool_body(h_hbm, batch_hbm, zeros_hbm, pool_a, pool_b,
               pacc, ts_b, ts_h, sem):
  c = lax.axis_index("c")
  s = lax.axis_index("s")
  w = c * NS + s
  gr = G_ACC // NS

  pltpu.sync_copy(zeros_hbm, pacc.at[pl.ds(s * gr, gr), :])
  pltpu.sync_copy(batch_hbm.at[w], ts_b)
  plsc.subcore_barrier()

  def chunk_body(j, carry):
    pltpu.sync_copy(h_hbm.at[pl.ds(w * 3200 + j * CHUNK, CHUNK), :], ts_h)
    pltpu.sync_copy(ts_h, pacc.at[ts_b.at[j]], add=True)
    return carry
  lax.fori_loop(0, 25, chunk_body, 0)

  plsc.subcore_barrier()

  @pl.when(c == 0)
  def _():
    pltpu.sync_copy(pacc.at[pl.ds(s * gr, gr), :],
                    pool_a.at[pl.ds(s * gr, gr), :])

  @pl.when(c == 1)
  def _():
    pltpu.sync_copy(pacc.at[pl.ds(s * gr, gr), :],
                    pool_b.at[pl.ds(s * gr, gr), :])


_pool_kernel = functools.partial(
    pl.kernel,
    _pool_body,
    out_type=[jax.ShapeDtypeStruct((G_ACC, 80), jnp.float32)] * 2,
    mesh=_mesh,
    compiler_params=pltpu.CompilerParams(use_tc_tiling_on_sc=False),
    scratch_types=[
        pltpu.VMEM_SHARED((G_ACC, 80), jnp.float32),
        pltpu.VMEM((32, CHUNK), jnp.int32),
        pltpu.VMEM((CHUNK, 80), jnp.float32),
        pltpu.SemaphoreType.DMA,
    ],
)()


def _prep_body(w_ref, m_ref, t_ref):
  t_ref[...] = w_ref[...] * m_ref[...]


def _dense0_body(p0, p1, p2a, p2b, e0, e1, e2, e3, f0, f1, f2, f3,
                 wmx, wme, wo, x1p0, x1p1, x1p2):
  a = (jnp.dot(p0[...], wmx[0:16, :], preferred_element_type=jnp.float32)
       + jnp.dot(p1[...], wmx[16:32, :], preferred_element_type=jnp.float32)
       + jnp.dot(p2a[...] + p2b[...], wmx[32:48, :],
                 preferred_element_type=jnp.float32)
       + (e0[...] + f0[...])[:, None] * wme[0:1, :]
       + (e1[...] + f1[...])[:, None] * wme[1:2, :]
       + (e2[...] + f2[...])[:, None] * wme[2:3, :]
       + (e3[...] + f3[...])[:, None] * wme[3:4, :])
  x1 = jnp.dot(_silu(a * INV_SQRT_NN), wo[...],
               preferred_element_type=jnp.float32)
  x1p0[...] = x1[:, 0:16]
  x1p1[...] = x1[:, 16:32]
  x1p2[...] = x1[:, 32:48]


def _dense1_body(p0, p1, p2a, p2b, e0, e1, e2, e3, f0, f1, f2, f3,
                 wmx, wme, wo, wsc, h_ext):
  a = (jnp.dot(p0[...], wmx[0:16, :], preferred_element_type=jnp.float32)
       + jnp.dot(p1[...], wmx[16:32, :], preferred_element_type=jnp.float32)
       + jnp.dot(p2a[...] + p2b[...], wmx[32:48, :],
                 preferred_element_type=jnp.float32)
       + (e0[...] + f0[...])[:, None] * wme[0:1, :]
       + (e1[...] + f1[...])[:, None] * wme[1:2, :]
       + (e2[...] + f2[...])[:, None] * wme[2:3, :]
       + (e3[...] + f3[...])[:, None] * wme[3:4, :])
  x2 = jnp.dot(_silu(a * INV_SQRT_NN), wo[...],
               preferred_element_type=jnp.float32)
  h = jnp.dot(x2, wsc[...], preferred_element_type=jnp.float32)
  rows = h.shape[0]
  h_ext[...] = jnp.concatenate(
      [h, jnp.ones((rows, 1), jnp.float32), jnp.zeros((rows, 15),
                                                      jnp.float32)],
      axis=1)


def _final_body(pa, pb, w1, b1, w2, b2, out):
  s = pa[...] + pb[...]
  counts = jnp.clip(s[:, HID:HID + 1], 1.0, None)
  hg = s[:, 0:HID] / counts
  z = _silu(jnp.dot(hg, w1[...], preferred_element_type=jnp.float32)
            + b1[...][None, :])
  o = jnp.dot(z, w2[...], preferred_element_type=jnp.float32) + b2[...][None, :]
  out[...] = o[0:G, 0]


def kernel(node_idx, edge_index, edge_attr, batch_vec, z,
           embed_W, embed_mask,
           Wm0x, Wm0e, Wo0, Wm1x, Wm1e, Wo1,
           Wsc, W1, b1, W2, b2):
  node_idx = node_idx.astype(jnp.int32)
  src = edge_index[0].astype(jnp.int32)
  dst = edge_index[1].astype(jnp.int32)
  batch_vec = batch_vec.astype(jnp.int32)

  ep = E_PAD - E
  pad_iota = lax.iota(jnp.int32, ep)
  src_pad = jnp.concatenate([src, pad_iota % N]).reshape(E_ROWS, CHUNK)
  dst_pad = jnp.concatenate([dst, N + (pad_iota % CHUNK)]).reshape(
      E_ROWS, CHUNK)
  ea_pad = jnp.concatenate(
      [edge_attr.T, jnp.zeros((EDGE, ep), jnp.float32)], axis=1
      ).reshape(EDGE * E_ROWS, CHUNK)
  np_ = N_PAD - N
  rep = (lax.iota(jnp.int32, N_PAD) // LANES) % 16 * NUM_TOKENS
  node_pad = jnp.pad(
      (jnp.concatenate([node_idx, jnp.zeros((np_,), jnp.int32)]) + rep
       ).reshape(32, 25, CHUNK), ((0, 0), (0, 7), (0, 0)))
  batch_pad = jnp.pad(
      jnp.concatenate([batch_vec, G + (lax.iota(jnp.int32, np_) % 8)]
                      ).reshape(32, 25, CHUNK),
      ((0, 0), (0, 7), (0, 0)), constant_values=G)
  zeros16 = jnp.zeros((N_ACC // NS, PANEL), jnp.float32)
  zeros1 = jnp.zeros((N_ACC // NS,), jnp.float32)
  zeros80 = jnp.zeros((G_ACC // NS, 80), jnp.float32)

  t_tab = pl.pallas_call(
      _prep_body,
      out_shape=jax.ShapeDtypeStruct((NUM_TOKENS, BASE), jnp.float32),
  )(embed_W, embed_mask)

  t0 = jnp.tile(t_tab[:, 0:16], (16, 1))
  t1 = jnp.tile(t_tab[:, 16:32], (16, 1))
  t2 = jnp.tile(t_tab[:, 32:48], (16, 1))
  (x0p0, x0p1, x0p2, ea_a0, ea_a1, ea_a2, ea_a3,
   ea_b0, ea_b1, ea_b2, ea_b3) = _embed_kernel(
      t0, t1, t2, node_pad, ea_pad, dst_pad, zeros1)

  src_pad_m = src_pad.reshape(E_PAD // 256, 256)
  dst_pad_m = dst_pad.reshape(E_PAD // 256, 256)
  a0p0, a0p1, a0p2a, a0p2b = _msg_kernel(
      src_pad_m, dst_pad_m, zeros16, x0p0, x0p1, x0p2)

  bs = 1024
  nblk = N_ACC // bs
  row_spec = pl.BlockSpec((bs, PANEL), lambda i: (i, 0))
  col_spec = pl.BlockSpec((bs,), lambda i: (i,))
  full = lambda shape: pl.BlockSpec(shape, lambda i: (0, 0))
  ea_args = (ea_a0, ea_a1, ea_a2, ea_a3, ea_b0, ea_b1, ea_b2, ea_b3)
  x1p0, x1p1, x1p2 = pl.pallas_call(
      _dense0_body,
      grid=(nblk,),
      in_specs=[row_spec] * 4 + [col_spec] * 8 + [
          full((BASE, 40)), full((EDGE, 40)), full((40, BASE))],
      out_specs=[row_spec] * 3,
      out_shape=[jax.ShapeDtypeStruct((N_ACC, PANEL), jnp.float32)] * 3,
  )(a0p0, a0p1, a0p2a, a0p2b, *ea_args, Wm0x, Wm0e, Wo0)

  a1p0, a1p1, a1p2a, a1p2b = _msg_kernel(
      src_pad_m, dst_pad_m, zeros16, x1p0, x1p1, x1p2)

  nblk1 = N_PAD // 1024
  h_ext = pl.pallas_call(
      _dense1_body,
      grid=(nblk1,),
      in_specs=[row_spec] * 4 + [col_spec] * 8 + [
          full((BASE, 80)), full((EDGE, 80)), full((80, 96)),
          full((96, HID))],
      out_specs=pl.BlockSpec((bs, 80), lambda i: (i, 0)),
      out_shape=jax.ShapeDtypeStruct((N_PAD, 80), jnp.float32),
  )(a1p0, a1p1, a1p2a, a1p2b, *ea_args, Wm1x, Wm1e, Wo1, Wsc)

  pool_a, pool_b = _pool_kernel(h_ext, batch_pad, zeros80)

  out = pl.pallas_call(
      _final_body,
      out_shape=jax.ShapeDtypeStruct((G,), jnp.float32),
  )(pool_a, pool_b, W1, b1, W2, b2)
  return out
